# baseline (speedup 1.0000x reference)
import jax
import jax.numpy as jnp
from jax.experimental import pallas as pl
from jax.experimental.pallas import tpu as pltpu

_TILE = 512


def _cheb_kernel(
    l_ref, x0t_ref, w_ref, b_ref, out_ref, lc_scr, x0_scr, x1_scr, x1b_scr, y_scr
):
    k = pl.program_id(0)
    r = pl.program_id(1)
    f = x0t_ref.shape[0]

    @pl.when(jnp.logical_and(k == 0, r == 0))
    def _transpose_x0():
        x0_scr[...] = jnp.transpose(x0t_ref[...], (1, 0))

    @pl.when(k == 0)
    def _first_pass():
        l_tile = l_ref[...]
        x1c = jnp.dot(l_tile, x0_scr[...], preferred_element_type=jnp.float32)
        row = r * _TILE
        x1_scr[pl.ds(row, _TILE), :] = x1c
        x1b = x1c.astype(jnp.bfloat16)
        x1b_scr[pl.ds(row, _TILE), :] = x1b
        lb_tile = l_tile.astype(jnp.bfloat16)
        lc_scr[pl.ds(row, _TILE), :] = lb_tile

        def yc_body(c, acc):
            return acc + jnp.dot(
                lc_scr[pl.ds(row, _TILE), pl.ds(c * _TILE, _TILE)],
                x1b_scr[pl.ds(c * _TILE, _TILE), :],
                preferred_element_type=jnp.float32,
            )
        y_scr[pl.ds(row, _TILE), :] = jax.lax.fori_loop(
            0, r + 1, yc_body, jnp.zeros((_TILE, f), jnp.float32)
        )

        def yr_body(i, carry):
            irow = i * _TILE
            y_scr[pl.ds(irow, _TILE), :] += jnp.dot(
                lc_scr[pl.ds(irow, _TILE), pl.ds(row, _TILE)],
                x1b,
                preferred_element_type=jnp.float32,
            )
            return carry

        jax.lax.fori_loop(0, r, yr_body, 0)

    @pl.when(k == 1)
    def _second_pass():
        row = r * _TILE
        y = y_scr[pl.ds(row, _TILE), :]
        x0_r = x0_scr[pl.ds(row, _TILE), :]
        x1_r = x1_scr[pl.ds(row, _TILE), :]
        x2_r = 2.0 * y - x0_r
        acc = jnp.dot(x0_r, w_ref[0:f, :], preferred_element_type=jnp.float32)
        acc += jnp.dot(x1_r, w_ref[f : 2 * f, :], preferred_element_type=jnp.float32)
        acc += jnp.dot(x2_r, w_ref[2 * f : 3 * f, :], preferred_element_type=jnp.float32)
        out_ref[...] = jnp.transpose(acc, (1, 0)) + b_ref[...]


def kernel(laplacian, inputs, weight, bias, precompute=0, einsum=0):
    B, Fin, V, X, Y, Z = inputs.shape
    K, _, Fout = weight.shape
    F = Fin * B * X * Y * Z

    x0t = inputs.reshape(F, V)
    w3 = weight.reshape(K * Fin, Fout)
    b2d = bias.reshape(Fout, 1)

    R = V // _TILE

    out_t = pl.pallas_call(
        _cheb_kernel,
        grid=(2, R),
        in_specs=[
            pl.BlockSpec((_TILE, V), lambda k, r: (jnp.where(k == 0, r, R - 1), 0)),
            pl.BlockSpec((F, V), lambda k, r: (0, 0)),
            pl.BlockSpec((K * F, Fout), lambda k, r: (0, 0)),
            pl.BlockSpec((Fout, 1), lambda k, r: (0, 0)),
        ],
        out_specs=pl.BlockSpec((Fout, _TILE), lambda k, r: (0, jnp.where(k == 1, r, 0))),
        out_shape=jax.ShapeDtypeStruct((Fout, V), jnp.float32),
        scratch_shapes=[
            pltpu.VMEM((V, V), jnp.bfloat16),
            pltpu.VMEM((V, F), jnp.float32),
            pltpu.VMEM((V, F), jnp.float32),
            pltpu.VMEM((V, F), jnp.bfloat16),
            pltpu.VMEM((V, F), jnp.float32),
        ],
    )(laplacian, x0t, w3, b2d)

    return out_t.reshape(B, Fout, V, X, Y, Z)
